# baseline (speedup 1.0000x reference)
import functools

import jax
import jax.numpy as jnp
from jax import lax
from jax.experimental import pallas as pl
from jax.experimental.pallas import tpu as pltpu

ND = 1024
BN = 1024
BIG_F = 3.0e38


def _body(nsteps, xyz_ref, oxyzt_ref, feat_ref, ofeat_ref, w0_ref, w1_ref,
          out_ref, sta_ref, stb_ref):
    i = pl.program_id(0)

    @pl.when(i < nsteps)
    def selection():
        k = xyz_ref[0]
        ut = oxyzt_ref[0]

        d2 = None
        for c in range(3):
            diff = k[:, c:c + 1] - ut[c:c + 1, :]
            d2 = diff * diff if d2 is None else d2 + diff * diff

        iota0 = lax.broadcasted_iota(jnp.int32, (ND, BN), 0)

        keyb = lax.bitcast_convert_type(d2, jnp.int32)
        key = lax.bitcast_convert_type(keyb | iota0, jnp.float32)

        kms = []
        hits = []
        for j in range(3):
            km = jnp.min(key, axis=0, keepdims=True)
            hit = key == km
            kms.append(km)
            hits.append(hit)
            if j < 2:
                key = jnp.where(hit, BIG_F, key)

        r0 = 1.0 / (kms[0] + 1e-8)
        r1 = 1.0 / (kms[1] + 1e-8)
        r2 = 1.0 / (kms[2] + 1e-8)
        rnorm = 1.0 / (r0 + r1 + r2)

        st = jnp.where(hits[0], r0 * rnorm, 0.0)
        st = jnp.where(hits[1], r1 * rnorm, st)
        st = jnp.where(hits[2], r2 * rnorm, st)
        st = st.astype(jnp.bfloat16)

        @pl.when(i % 2 == 0)
        def _():
            sta_ref[...] = st

        @pl.when(i % 2 == 1)
        def _():
            stb_ref[...] = st

    @pl.when(i > 0)
    def matmuls():
        even_prev = (i - 1) % 2 == 0
        st = jnp.where(even_prev, sta_ref[...], stb_ref[...])

        feat = feat_ref[0]
        interp = lax.dot_general(feat, st, (((1,), (0,)), ((), ())),
                                 preferred_element_type=jnp.float32)

        x = jnp.concatenate([interp.astype(jnp.bfloat16), ofeat_ref[0]],
                            axis=0)
        h = lax.dot_general(w0_ref[...], x, (((1,), (0,)), ((), ())),
                            preferred_element_type=jnp.float32)
        h = jnp.maximum(h, 0.0).astype(jnp.bfloat16)
        h = lax.dot_general(w1_ref[...], h, (((1,), (0,)), ((), ())),
                            preferred_element_type=jnp.float32)
        out_ref[0] = jnp.maximum(h, 0.0)


@functools.partial(jax.jit, static_argnames=("interpret",))
def _run(xyzs, original_xyzs, features, original_features, W0, W1,
         interpret=False):
    B, T, ND_, _ = xyzs.shape
    NO = original_xyzs.shape[2]
    CIN = features.shape[2]
    CORIG = original_features.shape[2]
    MLP1 = W1.shape[0]
    BT = B * T
    nob = NO // BN
    nsteps = BT * nob

    xyz_f = xyzs.reshape(BT, ND_, 3)
    oxyz_t = original_xyzs.reshape(BT, NO, 3).transpose(0, 2, 1)
    feat_f = features.reshape(BT, CIN, ND_).astype(jnp.bfloat16)
    ofeat_f = original_features.reshape(BT, CORIG, NO).astype(jnp.bfloat16)
    W0 = W0.astype(jnp.bfloat16)
    W1 = W1.astype(jnp.bfloat16)

    def sel_f(i):
        si = jnp.minimum(i, nsteps - 1)
        return si // nob
    def sel_n(i):
        si = jnp.minimum(i, nsteps - 1)
        return si % nob
    def mm_f(i):
        mi = jnp.maximum(i - 1, 0)
        return mi // nob
    def mm_n(i):
        mi = jnp.maximum(i - 1, 0)
        return mi % nob

    out = pl.pallas_call(
        functools.partial(_body, nsteps),
        grid=(nsteps + 1,),
        in_specs=[
            pl.BlockSpec((1, ND_, 3), lambda i: (sel_f(i), 0, 0)),
            pl.BlockSpec((1, 3, BN), lambda i: (sel_f(i), 0, sel_n(i))),
            pl.BlockSpec((1, CIN, ND_), lambda i: (mm_f(i), 0, 0)),
            pl.BlockSpec((1, CORIG, BN), lambda i: (mm_f(i), 0, mm_n(i))),
            pl.BlockSpec((W0.shape[0], W0.shape[1]), lambda i: (0, 0)),
            pl.BlockSpec((MLP1, W1.shape[1]), lambda i: (0, 0)),
        ],
        out_specs=pl.BlockSpec((1, MLP1, BN), lambda i: (mm_f(i), 0, mm_n(i))),
        out_shape=jax.ShapeDtypeStruct((BT, MLP1, NO), jnp.float32),
        scratch_shapes=[
            pltpu.VMEM((ND_, BN), jnp.bfloat16),
            pltpu.VMEM((ND_, BN), jnp.bfloat16),
        ],
        compiler_params=pltpu.CompilerParams(
            dimension_semantics=("arbitrary",),
        ),
        interpret=interpret,
    )(xyz_f, oxyz_t, feat_f, ofeat_f, W0, W1)

    return original_xyzs, out.reshape(B, T, MLP1, NO)


def kernel(xyzs, original_xyzs, features, original_features, W0, W1):
    return _run(xyzs, original_xyzs, features, original_features, W0, W1)
